# baseline (speedup 1.0000x reference)
import functools

import jax
import jax.numpy as jnp
from jax.experimental import pallas as pl
from jax.experimental.pallas import tpu as pltpu

N1, N2, D = 4096, 100000, 64
NQ = N1 // 2
TC = 1024
NT = N2 // TC
T1 = 512
T2 = 160

K2 = 1024
NT2 = N1 // K2


def _phase1_body(d0t_ref, d1_ref, rmax_ref, nn12_ref):
    j = pl.program_id(1)
    nk = d1_ref.shape[0]
    simt = jnp.dot(d1_ref[...], d0t_ref[...],
                   preferred_element_type=jnp.float32)
    rmax_t = jnp.max(simt, axis=0)
    kid = jax.lax.broadcasted_iota(jnp.int32, (nk, NQ), 0)
    rarg_t = j * TC + jnp.min(
        jnp.where(simt == rmax_t[None, :], kid, jnp.int32(2**31 - 1)),
        axis=0)

    @pl.when(j == 0)
    def _():
        rmax_ref[...] = rmax_t
        nn12_ref[...] = rarg_t

    @pl.when(j > 0)
    def _():
        prev = rmax_ref[...]
        upd = rmax_t > prev
        nn12_ref[...] = jnp.where(upd, rarg_t, nn12_ref[...])
        rmax_ref[...] = jnp.where(upd, rmax_t, prev)


def _phase2_body(sel_ref, d0_ref, carg_ref):
    sim2 = jax.lax.dot_general(d0_ref[...], sel_ref[...],
                               (((1,), (1,)), ((), ())),
                               preferred_element_type=jnp.float32)
    cmax = jnp.max(sim2, axis=0)
    qid = jax.lax.broadcasted_iota(jnp.int32, (N1, K2), 0)
    carg = jnp.min(jnp.where(sim2 == cmax[None, :], qid, jnp.int32(2**31 - 1)),
                   axis=0)
    carg_ref[...] = carg


def _row_pass(d0t, d1, block, index_map, grid):
    return pl.pallas_call(
        _phase1_body,
        grid=(2, grid),
        in_specs=[
            pl.BlockSpec((D, NQ), lambda q, j: (0, q)),
            pl.BlockSpec((block, D), lambda q, j: index_map(j)),
        ],
        out_specs=[
            pl.BlockSpec((NQ,), lambda q, j: (q,)),
            pl.BlockSpec((NQ,), lambda q, j: (q,)),
        ],
        out_shape=[
            jax.ShapeDtypeStruct((N1,), jnp.float32),
            jax.ShapeDtypeStruct((N1,), jnp.int32),
        ],
        compiler_params=pltpu.CompilerParams(
            dimension_semantics=("parallel", "arbitrary")),
    )(d0t, d1)


@functools.partial(jax.jit)
def _matcher(d0, d1):
    d0t = d0.T
    rmax, nn12 = _row_pass(d0t, d1, TC, lambda j: (j, 0), NT)
    parts = [
        _row_pass(d0t, d1, T1, lambda j: (NT * TC // T1, 0), 1),
        _row_pass(d0t, d1, T2, lambda j: ((NT * TC + T1) // T2, 0), 1),
    ]
    for base, (rmax_p, nn12_p) in zip((NT * TC, NT * TC + T1), parts):
        upd = rmax_p > rmax
        nn12 = jnp.where(upd, nn12_p + base, nn12)
        rmax = jnp.where(upd, rmax_p, rmax)
    nn12 = nn12.astype(jnp.int32)

    sel = jnp.take(d1, nn12, axis=0)

    nn21_sel = pl.pallas_call(
        _phase2_body,
        grid=(NT2,),
        in_specs=[
            pl.BlockSpec((K2, D), lambda j: (j, 0)),
            pl.BlockSpec((N1, D), lambda j: (0, 0)),
        ],
        out_specs=pl.BlockSpec((K2,), lambda j: (j,)),
        out_shape=jax.ShapeDtypeStruct((N1,), jnp.int32),
        compiler_params=pltpu.CompilerParams(
            dimension_semantics=("parallel",)),
    )(sel, d0)

    mutual = jnp.arange(N1, dtype=jnp.int32) == nn21_sel
    all_matches = jnp.where(mutual, nn12, -1).astype(jnp.int64)
    return all_matches, rmax


def kernel(descriptors0, descriptors1):
    return _matcher(descriptors0, descriptors1)


# speedup vs baseline: 1.0781x; 1.0781x over previous
import functools

import jax
import jax.numpy as jnp
from jax.experimental import pallas as pl

N1, N2, D = 4096, 100000, 64
TC = 1024
NT = N2 // TC
T1 = 512
T2 = 160

K2 = 1024
NT2 = N1 // K2


def _phase1_body(d0t_ref, d1_ref, rmax_ref, nn12_ref):
    j = pl.program_id(0)
    nk = d1_ref.shape[0]
    simt = jnp.dot(d1_ref[...], d0t_ref[...],
                   preferred_element_type=jnp.float32)
    rmax_t = jnp.max(simt, axis=0)
    kid = jax.lax.broadcasted_iota(jnp.int32, (nk, N1), 0)
    rarg_t = j * TC + jnp.min(
        jnp.where(simt == rmax_t[None, :], kid, jnp.int32(2**31 - 1)),
        axis=0)

    @pl.when(j == 0)
    def _():
        rmax_ref[...] = rmax_t
        nn12_ref[...] = rarg_t

    @pl.when(j > 0)
    def _():
        prev = rmax_ref[...]
        upd = rmax_t > prev
        nn12_ref[...] = jnp.where(upd, rarg_t, nn12_ref[...])
        rmax_ref[...] = jnp.where(upd, rmax_t, prev)


def _phase2_body(sel_ref, d0_ref, carg_ref):
    sim2 = jax.lax.dot_general(d0_ref[...], sel_ref[...],
                               (((1,), (1,)), ((), ())),
                               preferred_element_type=jnp.float32)
    cmax = jnp.max(sim2, axis=0)
    qid = jax.lax.broadcasted_iota(jnp.int32, (N1, K2), 0)
    carg = jnp.min(jnp.where(sim2 == cmax[None, :], qid, jnp.int32(2**31 - 1)),
                   axis=0)
    carg_ref[...] = carg


def _row_pass(d0t, d1, block, index_map, grid):
    return pl.pallas_call(
        _phase1_body,
        grid=(grid,),
        in_specs=[
            pl.BlockSpec((D, N1), lambda j: (0, 0)),
            pl.BlockSpec((block, D), index_map),
        ],
        out_specs=[
            pl.BlockSpec((N1,), lambda j: (0,)),
            pl.BlockSpec((N1,), lambda j: (0,)),
        ],
        out_shape=[
            jax.ShapeDtypeStruct((N1,), jnp.float32),
            jax.ShapeDtypeStruct((N1,), jnp.int32),
        ],
    )(d0t, d1)


@functools.partial(jax.jit)
def _matcher(d0, d1):
    d0t = d0.T
    rmax, nn12 = _row_pass(d0t, d1, TC, lambda j: (j, 0), NT)
    parts = [
        _row_pass(d0t, d1, T1, lambda j: (NT * TC // T1, 0), 1),
        _row_pass(d0t, d1, T2, lambda j: ((NT * TC + T1) // T2, 0), 1),
    ]
    for base, (rmax_p, nn12_p) in zip((NT * TC, NT * TC + T1), parts):
        upd = rmax_p > rmax
        nn12 = jnp.where(upd, nn12_p + base, nn12)
        rmax = jnp.where(upd, rmax_p, rmax)
    nn12 = nn12.astype(jnp.int32)

    sel = jnp.take(d1, nn12, axis=0)

    nn21_sel = pl.pallas_call(
        _phase2_body,
        grid=(NT2,),
        in_specs=[
            pl.BlockSpec((K2, D), lambda j: (j, 0)),
            pl.BlockSpec((N1, D), lambda j: (0, 0)),
        ],
        out_specs=pl.BlockSpec((K2,), lambda j: (j,)),
        out_shape=jax.ShapeDtypeStruct((N1,), jnp.int32),
    )(sel, d0)

    mutual = jnp.arange(N1, dtype=jnp.int32) == nn21_sel
    all_matches = jnp.where(mutual, nn12, -1).astype(jnp.int64)
    return all_matches, rmax


def kernel(descriptors0, descriptors1):
    return _matcher(descriptors0, descriptors1)
